# baseline (speedup 1.0000x reference)
import functools

import jax
import jax.numpy as jnp
from jax import lax
from jax.experimental import pallas as pl
from jax.experimental.pallas import tpu as pltpu
from jax.experimental.pallas import tpu_sc as plsc

N_NODES = 4096
N_EDGES = 262144
FEAT = 128

NC = 2
NS = 16
NW = NC * NS
EPW = N_EDGES // NW
CHUNK = 128
NCHUNK = EPW // CHUNK

_SELU_SCALE = 1.0507009873554804934193349852946
_SELU_ALPHA = 1.6732632423543772848170429916717


def _selu(x):
    return _SELU_SCALE * jnp.where(x > 0, x, _SELU_ALPHA * (jnp.exp(x) - 1.0))


def _sc_mesh():
    return plsc.VectorSubcoreMesh(
        core_axis_name="c", subcore_axis_name="s", num_cores=NC, num_subcores=NS
    )


def _sc_gather_bits(hf, senders, receivers):
    @functools.partial(
        pl.kernel,
        out_type=(
            jax.ShapeDtypeStruct((N_EDGES,), jnp.float32),
            jax.ShapeDtypeStruct((N_EDGES,), jnp.float32),
        ),
        mesh=_sc_mesh(),
        compiler_params=pltpu.CompilerParams(needs_layout_passes=False),
        scratch_types=[
            pltpu.VMEM((N_NODES,), jnp.float32),
            pltpu.VMEM((EPW,), jnp.int32),
            pltpu.VMEM((EPW,), jnp.int32),
            pltpu.VMEM((EPW,), jnp.float32),
            pltpu.VMEM((EPW,), jnp.float32),
        ],
    )
    def k(hf_hbm, snd_hbm, rcv_hbm, rb_hbm, sb_hbm, hf_v, si_v, ri_v, rb_v, sb_v):
        wid = lax.axis_index("s") * NC + lax.axis_index("c")
        base = wid * EPW
        pltpu.sync_copy(hf_hbm, hf_v)
        pltpu.sync_copy(snd_hbm.at[pl.ds(base, EPW)], si_v)
        pltpu.sync_copy(rcv_hbm.at[pl.ds(base, EPW)], ri_v)

        def body(i, _):
            for u in range(8):
                off = i * 128 + u * 16
                ridx = ri_v[pl.ds(off, 16)]
                sidx = si_v[pl.ds(off, 16)]
                rb_v[pl.ds(off, 16)] = plsc.load_gather(hf_v, [ridx])
                sb_v[pl.ds(off, 16)] = plsc.load_gather(hf_v, [sidx])
            return 0

        lax.fori_loop(0, EPW // 128, body, 0)
        pltpu.sync_copy(rb_v, rb_hbm.at[pl.ds(base, EPW)])
        pltpu.sync_copy(sb_v, sb_hbm.at[pl.ds(base, EPW)])

    return k(hf, senders, receivers)


def _selu_lean(t):
    sa = _SELU_SCALE * _SELU_ALPHA
    neg = sa * jnp.exp(t) - sa
    return jnp.where(t > 0, _SELU_SCALE * t, neg)


def _tc_messages(rb, sb, w, coef, W2p, b2, off, nblocks):
    BE = 4096
    ne = nblocks * BE
    grid = (nblocks,)

    def body(rb_ref, sb_ref, w_ref, coef_ref, w2_ref, b2_ref, out_ref):
        cf = coef_ref[...]
        c0 = cf[0, :16][:, None]
        dr = cf[1, :16][:, None]
        ds = cf[2, :16][:, None]
        vv = cf[3, :16][:, None]
        rbb = rb_ref[...][None, :]
        sbb = sb_ref[...][None, :]
        ww = w_ref[...][None, :]
        x_pre = c0 + dr * rbb + ds * sbb + vv * ww
        xT = _selu_lean(x_pre)
        y = jax.lax.dot_general(
            xT, w2_ref[...], (((0,), (0,)), ((), ())),
            preferred_element_type=jnp.float32,
        )
        out_ref[...] = _selu_lean(y + b2_ref[...][None, :])

    return pl.pallas_call(
        body,
        grid=grid,
        in_specs=[
            pl.BlockSpec((BE,), lambda i: (i + off,)),
            pl.BlockSpec((BE,), lambda i: (i + off,)),
            pl.BlockSpec((BE,), lambda i: (i + off,)),
            pl.BlockSpec((8, 128), lambda i: (0, 0)),
            pl.BlockSpec((16, FEAT), lambda i: (0, 0)),
            pl.BlockSpec((FEAT,), lambda i: (0,)),
        ],
        out_specs=pl.BlockSpec((BE, FEAT), lambda i: (i, 0)),
        out_shape=jax.ShapeDtypeStruct((ne, FEAT), jnp.float32),
        compiler_params=pltpu.CompilerParams(fuse_transposed_lhs_in_matmul=True),
    )(rb, sb, w, coef, W2p, b2)


def _sc_scatter(msg, recv3d, zeros, ne):
    ROWS = N_NODES // NS
    epw = ne // NW
    nchunk = epw // CHUNK

    NBUF = 5
    AHEAD = 2
    INFLIGHT = 3

    @functools.partial(
        pl.kernel,
        out_type=jax.ShapeDtypeStruct((NC * N_NODES, FEAT), jnp.float32),
        mesh=_sc_mesh(),
        scratch_types=[
            pltpu.VMEM((nchunk, CHUNK), jnp.int32),
            pltpu.VMEM((NBUF, CHUNK, FEAT), jnp.float32),
            pltpu.VMEM_SHARED((N_NODES, FEAT), jnp.float32),
            [pltpu.SemaphoreType.DMA] * NBUF,
            [pltpu.SemaphoreType.DMA] * NBUF,
            pltpu.SemaphoreType.DMA,
        ],
    )
    def k(msg_hbm, idx_hbm, zero_hbm, out_hbm, idx_v, msg_v,
          agg_sh, fsems, ssems, wsem):
        c = lax.axis_index("c")
        s = lax.axis_index("s")
        wid = s * NC + c
        base = wid * epw

        def fetch(j):
            return pltpu.async_copy(
                msg_hbm.at[pl.ds(base + j * CHUNK, CHUNK)],
                msg_v.at[j % NBUF], fsems[j % NBUF])

        def scat(j):
            return pltpu.async_copy(
                msg_v.at[j % NBUF], agg_sh.at[idx_v.at[j]],
                ssems[j % NBUF], add=True)

        fds = [None] * NBUF
        sds = [None] * NBUF
        for j in range(AHEAD):
            fds[j] = fetch(j)
        idx_cp = pltpu.async_copy(idx_hbm.at[wid], idx_v, wsem)

        pltpu.sync_copy(zero_hbm, msg_v.at[NBUF - 1])
        pltpu.sync_copy(msg_v.at[NBUF - 1], agg_sh.at[pl.ds(s * ROWS, CHUNK)])
        pltpu.sync_copy(msg_v.at[NBUF - 1],
                        agg_sh.at[pl.ds(s * ROWS + CHUNK, CHUNK)])
        idx_cp.wait()
        plsc.subcore_barrier()

        for j in range(nchunk):
            fds[j % NBUF].wait()
            sds[j % NBUF] = scat(j)
            if j >= INFLIGHT:
                sds[(j - INFLIGHT) % NBUF].wait()
            if j + AHEAD < nchunk:
                fds[(j + AHEAD) % NBUF] = fetch(j + AHEAD)
        for j in range(max(nchunk - INFLIGHT, 0), nchunk):
            sds[j % NBUF].wait()
        plsc.subcore_barrier()

        obase = c * N_NODES + s * ROWS
        pltpu.sync_copy(agg_sh.at[pl.ds(s * ROWS, CHUNK)], msg_v.at[0])
        w0 = pltpu.async_copy(msg_v.at[0], out_hbm.at[pl.ds(obase, CHUNK)],
                              fsems[0])
        pltpu.sync_copy(agg_sh.at[pl.ds(s * ROWS + CHUNK, CHUNK)], msg_v.at[1])
        pltpu.sync_copy(msg_v.at[1], out_hbm.at[pl.ds(obase + CHUNK, CHUNK)])
        w0.wait()

    return k(msg, recv3d, zeros)


def _tc_readout(p0, p1, W_a1, b_a1, W_a2, b_a2):
    BJ = 512
    NB = N_NODES // BJ
    grid = (2 * NB,)

    def body(p0_ref, p1_ref, w1_ref, b1_ref, w2_ref, b2_ref, out_ref,
             hs_ref, z_ref):
        i = pl.program_id(0)

        @pl.when(i == 0)
        def _():
            agg = (p0_ref[: N_NODES, :] + p0_ref[N_NODES:, :]
                   + p1_ref[: N_NODES, :] + p1_ref[N_NODES:, :])
            hs_ref[...] = jnp.sum(jnp.maximum(agg, 0.0), axis=1)
            out_ref[...] = jnp.zeros((1, 1), jnp.float32)

        @pl.when(i < NB)
        def _():
            y = jnp.dot(hs_ref[...][None, :], w1_ref[...],
                        preferred_element_type=jnp.float32)
            z_ref[i, :] = _selu_lean(y[0] + b1_ref[...])

        @pl.when(i >= NB)
        def _():
            y = jnp.zeros((1, BJ), jnp.float32)
            for kk in range(NB):
                y += jnp.dot(z_ref[kk, :][None, :],
                             w2_ref[pl.ds(kk * BJ, BJ), :],
                             preferred_element_type=jnp.float32)
            zz = _selu_lean(y[0] + b2_ref[...])
            out_ref[...] += jnp.sum(zz).reshape(1, 1)

    return pl.pallas_call(
        body,
        grid=grid,
        in_specs=[
            pl.BlockSpec((NC * N_NODES, FEAT), lambda i: (0, 0)),
            pl.BlockSpec((NC * N_NODES, FEAT), lambda i: (0, 0)),
            pl.BlockSpec((N_NODES, BJ), lambda i: (0, jnp.minimum(i, NB - 1))),
            pl.BlockSpec((BJ,), lambda i: (jnp.minimum(i, NB - 1),)),
            pl.BlockSpec((N_NODES, BJ), lambda i: (0, jnp.maximum(i - NB, 0))),
            pl.BlockSpec((BJ,), lambda i: (jnp.maximum(i - NB, 0),)),
        ],
        out_specs=pl.BlockSpec((1, 1), lambda i: (0, 0)),
        out_shape=jax.ShapeDtypeStruct((1, 1), jnp.float32),
        scratch_shapes=[
            pltpu.VMEM((N_NODES,), jnp.float32),
            pltpu.VMEM((NB, BJ), jnp.float32),
        ],
    )(p0, p1, W_a1, b_a1, W_a2, b_a2)


def kernel(h, senders, receivers, edge_weights, embed, W_m1, b_m1, W_m2, b_m2,
           Wq, bq, Wk, bk, W_a1, b_a1, W_a2, b_a2):
    A = W_m1[:5]
    B = W_m1[5:10]
    v = W_m1[10]
    e0 = embed[0]
    e1 = embed[1]
    c0 = e0 @ A + e0 @ B + b_m1
    dr = (e1 - e0) @ A
    ds = (e1 - e0) @ B

    coef = jnp.zeros((8, 128), jnp.float32)
    coef = coef.at[0, :11].set(c0)
    coef = coef.at[1, :11].set(dr)
    coef = coef.at[2, :11].set(ds)
    coef = coef.at[3, :11].set(v)
    W2p = jnp.zeros((16, FEAT), jnp.float32).at[:11, :].set(W_m2)

    hf = h.astype(jnp.float32)
    snd = senders.astype(jnp.int32)
    rcv = receivers.astype(jnp.int32)

    rb, sb = _sc_gather_bits(hf, snd, rcv)
    SPLITS = (26, 38)
    zeros = jnp.zeros((CHUNK, FEAT), jnp.float32)
    parts = []
    off = 0
    for nb in SPLITS:
        ne = nb * 4096
        msg_h = _tc_messages(rb, sb, edge_weights, coef, W2p, b_m2, off, nb)
        rcv_h = lax.dynamic_slice_in_dim(rcv, off * 4096, ne)
        parts.append(_sc_scatter(
            msg_h, rcv_h.reshape(NW, ne // NW // CHUNK, CHUNK), zeros, ne))
        off += nb
    res = _tc_readout(parts[0], parts[1], W_a1, b_a1, W_a2, b_a2)
    return res[0, 0]


# speedup vs baseline: 1.0381x; 1.0381x over previous
import functools

import jax
import jax.numpy as jnp
from jax import lax
from jax.experimental import pallas as pl
from jax.experimental.pallas import tpu as pltpu
from jax.experimental.pallas import tpu_sc as plsc

N_NODES = 4096
N_EDGES = 262144
FEAT = 128

NC = 2
NS = 16
NW = NC * NS
EPW = N_EDGES // NW
CHUNK = 128
NCHUNK = EPW // CHUNK

_SELU_SCALE = 1.0507009873554804934193349852946
_SELU_ALPHA = 1.6732632423543772848170429916717


def _selu(x):
    return _SELU_SCALE * jnp.where(x > 0, x, _SELU_ALPHA * (jnp.exp(x) - 1.0))


def _sc_mesh():
    return plsc.VectorSubcoreMesh(
        core_axis_name="c", subcore_axis_name="s", num_cores=NC, num_subcores=NS
    )


def _sc_gather_bits(hf, senders, receivers):
    @functools.partial(
        pl.kernel,
        out_type=(
            jax.ShapeDtypeStruct((N_EDGES,), jnp.float32),
            jax.ShapeDtypeStruct((N_EDGES,), jnp.float32),
        ),
        mesh=_sc_mesh(),
        compiler_params=pltpu.CompilerParams(needs_layout_passes=False),
        scratch_types=[
            pltpu.VMEM((N_NODES,), jnp.float32),
            pltpu.VMEM((EPW,), jnp.int32),
            pltpu.VMEM((EPW,), jnp.int32),
            pltpu.VMEM((EPW,), jnp.float32),
            pltpu.VMEM((EPW,), jnp.float32),
        ],
    )
    def k(hf_hbm, snd_hbm, rcv_hbm, rb_hbm, sb_hbm, hf_v, si_v, ri_v, rb_v, sb_v):
        wid = lax.axis_index("s") * NC + lax.axis_index("c")
        base = wid * EPW
        pltpu.sync_copy(hf_hbm, hf_v)
        pltpu.sync_copy(snd_hbm.at[pl.ds(base, EPW)], si_v)
        pltpu.sync_copy(rcv_hbm.at[pl.ds(base, EPW)], ri_v)

        def body(i, _):
            for u in range(8):
                off = i * 128 + u * 16
                ridx = ri_v[pl.ds(off, 16)]
                sidx = si_v[pl.ds(off, 16)]
                rb_v[pl.ds(off, 16)] = plsc.load_gather(hf_v, [ridx])
                sb_v[pl.ds(off, 16)] = plsc.load_gather(hf_v, [sidx])
            return 0

        lax.fori_loop(0, EPW // 128, body, 0)
        pltpu.sync_copy(rb_v, rb_hbm.at[pl.ds(base, EPW)])
        pltpu.sync_copy(sb_v, sb_hbm.at[pl.ds(base, EPW)])

    return k(hf, senders, receivers)


def _selu_lean(t):
    sa = _SELU_SCALE * _SELU_ALPHA
    neg = sa * jnp.exp(t) - sa
    return jnp.where(t > 0, _SELU_SCALE * t, neg)


def _tc_messages(rb, sb, w, coef, W2p, b2, off, nblocks):
    BE = 4096
    ne = nblocks * BE
    grid = (nblocks,)

    def body(rb_ref, sb_ref, w_ref, coef_ref, w2_ref, b2_ref, out_ref):
        cf = coef_ref[...]
        c0 = cf[0, :16][:, None]
        dr = cf[1, :16][:, None]
        ds = cf[2, :16][:, None]
        vv = cf[3, :16][:, None]
        rbb = rb_ref[...][None, :]
        sbb = sb_ref[...][None, :]
        ww = w_ref[...][None, :]
        x_pre = c0 + dr * rbb + ds * sbb + vv * ww
        xT = _selu_lean(x_pre)
        y = jax.lax.dot_general(
            xT, w2_ref[...], (((0,), (0,)), ((), ())),
            preferred_element_type=jnp.float32,
        )
        out_ref[...] = _selu_lean(y + b2_ref[...][None, :])

    return pl.pallas_call(
        body,
        grid=grid,
        in_specs=[
            pl.BlockSpec((BE,), lambda i: (i + off,)),
            pl.BlockSpec((BE,), lambda i: (i + off,)),
            pl.BlockSpec((BE,), lambda i: (i + off,)),
            pl.BlockSpec((8, 128), lambda i: (0, 0)),
            pl.BlockSpec((16, FEAT), lambda i: (0, 0)),
            pl.BlockSpec((FEAT,), lambda i: (0,)),
        ],
        out_specs=pl.BlockSpec((BE, FEAT), lambda i: (i, 0)),
        out_shape=jax.ShapeDtypeStruct((ne, FEAT), jnp.float32),
        compiler_params=pltpu.CompilerParams(fuse_transposed_lhs_in_matmul=True),
    )(rb, sb, w, coef, W2p, b2)


def _sc_scatter(msg, recv3d, zeros, ne):
    ROWS = N_NODES // NS
    epw = ne // NW
    nchunk = epw // CHUNK

    NBUF = 5
    AHEAD = 3
    INFLIGHT = 2

    @functools.partial(
        pl.kernel,
        out_type=jax.ShapeDtypeStruct((NC * N_NODES, FEAT), jnp.float32),
        mesh=_sc_mesh(),
        scratch_types=[
            pltpu.VMEM((nchunk, CHUNK), jnp.int32),
            pltpu.VMEM((NBUF, CHUNK, FEAT), jnp.float32),
            pltpu.VMEM_SHARED((N_NODES, FEAT), jnp.float32),
            [pltpu.SemaphoreType.DMA] * NBUF,
            [pltpu.SemaphoreType.DMA] * NBUF,
            pltpu.SemaphoreType.DMA,
        ],
    )
    def k(msg_hbm, idx_hbm, zero_hbm, out_hbm, idx_v, msg_v,
          agg_sh, fsems, ssems, wsem):
        c = lax.axis_index("c")
        s = lax.axis_index("s")
        wid = s * NC + c
        base = wid * epw

        def fetch(j):
            return pltpu.async_copy(
                msg_hbm.at[pl.ds(base + j * CHUNK, CHUNK)],
                msg_v.at[j % NBUF], fsems[j % NBUF])

        def scat(j):
            return pltpu.async_copy(
                msg_v.at[j % NBUF], agg_sh.at[idx_v.at[j]],
                ssems[j % NBUF], add=True)

        fds = [None] * NBUF
        sds = [None] * NBUF
        for j in range(AHEAD):
            fds[j] = fetch(j)
        idx_cp = pltpu.async_copy(idx_hbm.at[wid], idx_v, wsem)

        pltpu.sync_copy(zero_hbm, msg_v.at[NBUF - 1])
        pltpu.sync_copy(msg_v.at[NBUF - 1], agg_sh.at[pl.ds(s * ROWS, CHUNK)])
        pltpu.sync_copy(msg_v.at[NBUF - 1],
                        agg_sh.at[pl.ds(s * ROWS + CHUNK, CHUNK)])
        idx_cp.wait()
        plsc.subcore_barrier()

        for j in range(nchunk):
            fds[j % NBUF].wait()
            sds[j % NBUF] = scat(j)
            if j >= INFLIGHT:
                sds[(j - INFLIGHT) % NBUF].wait()
            if j + AHEAD < nchunk:
                fds[(j + AHEAD) % NBUF] = fetch(j + AHEAD)
        for j in range(max(nchunk - INFLIGHT, 0), nchunk):
            sds[j % NBUF].wait()
        plsc.subcore_barrier()

        obase = c * N_NODES + s * ROWS
        pltpu.sync_copy(agg_sh.at[pl.ds(s * ROWS, CHUNK)], msg_v.at[0])
        w0 = pltpu.async_copy(msg_v.at[0], out_hbm.at[pl.ds(obase, CHUNK)],
                              fsems[0])
        pltpu.sync_copy(agg_sh.at[pl.ds(s * ROWS + CHUNK, CHUNK)], msg_v.at[1])
        pltpu.sync_copy(msg_v.at[1], out_hbm.at[pl.ds(obase + CHUNK, CHUNK)])
        w0.wait()

    return k(msg, recv3d, zeros)


def _tc_readout(p0, p1, W_a1, b_a1, W_a2, b_a2):
    BJ = 512
    NB = N_NODES // BJ
    grid = (2 * NB,)

    def body(p0_ref, p1_ref, w1_ref, b1_ref, w2_ref, b2_ref, out_ref,
             hs_ref, z_ref):
        i = pl.program_id(0)

        @pl.when(i == 0)
        def _():
            agg = (p0_ref[: N_NODES, :] + p0_ref[N_NODES:, :]
                   + p1_ref[: N_NODES, :] + p1_ref[N_NODES:, :])
            hs_ref[...] = jnp.sum(jnp.maximum(agg, 0.0), axis=1)
            out_ref[...] = jnp.zeros((1, 1), jnp.float32)

        @pl.when(i < NB)
        def _():
            y = jnp.dot(hs_ref[...][None, :], w1_ref[...],
                        preferred_element_type=jnp.float32)
            z_ref[i, :] = _selu_lean(y[0] + b1_ref[...])

        @pl.when(i >= NB)
        def _():
            y = jnp.zeros((1, BJ), jnp.float32)
            for kk in range(NB):
                y += jnp.dot(z_ref[kk, :][None, :],
                             w2_ref[pl.ds(kk * BJ, BJ), :],
                             preferred_element_type=jnp.float32)
            zz = _selu_lean(y[0] + b2_ref[...])
            out_ref[...] += jnp.sum(zz).reshape(1, 1)

    return pl.pallas_call(
        body,
        grid=grid,
        in_specs=[
            pl.BlockSpec((NC * N_NODES, FEAT), lambda i: (0, 0)),
            pl.BlockSpec((NC * N_NODES, FEAT), lambda i: (0, 0)),
            pl.BlockSpec((N_NODES, BJ), lambda i: (0, jnp.minimum(i, NB - 1))),
            pl.BlockSpec((BJ,), lambda i: (jnp.minimum(i, NB - 1),)),
            pl.BlockSpec((N_NODES, BJ), lambda i: (0, jnp.maximum(i - NB, 0))),
            pl.BlockSpec((BJ,), lambda i: (jnp.maximum(i - NB, 0),)),
        ],
        out_specs=pl.BlockSpec((1, 1), lambda i: (0, 0)),
        out_shape=jax.ShapeDtypeStruct((1, 1), jnp.float32),
        scratch_shapes=[
            pltpu.VMEM((N_NODES,), jnp.float32),
            pltpu.VMEM((NB, BJ), jnp.float32),
        ],
    )(p0, p1, W_a1, b_a1, W_a2, b_a2)


def kernel(h, senders, receivers, edge_weights, embed, W_m1, b_m1, W_m2, b_m2,
           Wq, bq, Wk, bk, W_a1, b_a1, W_a2, b_a2):
    A = W_m1[:5]
    B = W_m1[5:10]
    v = W_m1[10]
    e0 = embed[0]
    e1 = embed[1]
    c0 = e0 @ A + e0 @ B + b_m1
    dr = (e1 - e0) @ A
    ds = (e1 - e0) @ B

    coef = jnp.zeros((8, 128), jnp.float32)
    coef = coef.at[0, :11].set(c0)
    coef = coef.at[1, :11].set(dr)
    coef = coef.at[2, :11].set(ds)
    coef = coef.at[3, :11].set(v)
    W2p = jnp.zeros((16, FEAT), jnp.float32).at[:11, :].set(W_m2)

    hf = h.astype(jnp.float32)
    snd = senders.astype(jnp.int32)
    rcv = receivers.astype(jnp.int32)

    rb, sb = _sc_gather_bits(hf, snd, rcv)
    SPLITS = (26, 38)
    zeros = jnp.zeros((CHUNK, FEAT), jnp.float32)
    parts = []
    off = 0
    for nb in SPLITS:
        ne = nb * 4096
        msg_h = _tc_messages(rb, sb, edge_weights, coef, W2p, b_m2, off, nb)
        rcv_h = lax.dynamic_slice_in_dim(rcv, off * 4096, ne)
        parts.append(_sc_scatter(
            msg_h, rcv_h.reshape(NW, ne // NW // CHUNK, CHUNK), zeros, ne))
        off += nb
    res = _tc_readout(parts[0], parts[1], W_a1, b_a1, W_a2, b_a2)
    return res[0, 0]


# speedup vs baseline: 1.0454x; 1.0070x over previous
import functools

import jax
import jax.numpy as jnp
from jax import lax
from jax.experimental import pallas as pl
from jax.experimental.pallas import tpu as pltpu
from jax.experimental.pallas import tpu_sc as plsc

N_NODES = 4096
N_EDGES = 262144
FEAT = 128

NC = 2
NS = 16
NW = NC * NS
EPW = N_EDGES // NW
CHUNK = 128
NCHUNK = EPW // CHUNK

_SELU_SCALE = 1.0507009873554804934193349852946
_SELU_ALPHA = 1.6732632423543772848170429916717


def _selu(x):
    return _SELU_SCALE * jnp.where(x > 0, x, _SELU_ALPHA * (jnp.exp(x) - 1.0))


def _sc_mesh():
    return plsc.VectorSubcoreMesh(
        core_axis_name="c", subcore_axis_name="s", num_cores=NC, num_subcores=NS
    )


def _sc_gather_bits(hf, senders, receivers):
    @functools.partial(
        pl.kernel,
        out_type=(
            jax.ShapeDtypeStruct((N_EDGES,), jnp.float32),
            jax.ShapeDtypeStruct((N_EDGES,), jnp.float32),
        ),
        mesh=_sc_mesh(),
        compiler_params=pltpu.CompilerParams(needs_layout_passes=False),
        scratch_types=[
            pltpu.VMEM((N_NODES,), jnp.float32),
            pltpu.VMEM((EPW,), jnp.int32),
            pltpu.VMEM((EPW,), jnp.int32),
            pltpu.VMEM((EPW,), jnp.float32),
            pltpu.VMEM((EPW,), jnp.float32),
        ],
    )
    def k(hf_hbm, snd_hbm, rcv_hbm, rb_hbm, sb_hbm, hf_v, si_v, ri_v, rb_v, sb_v):
        wid = lax.axis_index("s") * NC + lax.axis_index("c")
        base = wid * EPW
        pltpu.sync_copy(hf_hbm, hf_v)
        pltpu.sync_copy(snd_hbm.at[pl.ds(base, EPW)], si_v)
        pltpu.sync_copy(rcv_hbm.at[pl.ds(base, EPW)], ri_v)

        def body(i, _):
            for u in range(8):
                off = i * 128 + u * 16
                ridx = ri_v[pl.ds(off, 16)]
                sidx = si_v[pl.ds(off, 16)]
                rb_v[pl.ds(off, 16)] = plsc.load_gather(hf_v, [ridx])
                sb_v[pl.ds(off, 16)] = plsc.load_gather(hf_v, [sidx])
            return 0

        lax.fori_loop(0, EPW // 128, body, 0)
        pltpu.sync_copy(rb_v, rb_hbm.at[pl.ds(base, EPW)])
        pltpu.sync_copy(sb_v, sb_hbm.at[pl.ds(base, EPW)])

    return k(hf, senders, receivers)


def _selu_lean(t):
    sa = _SELU_SCALE * _SELU_ALPHA
    neg = sa * jnp.exp(t) - sa
    return jnp.where(t > 0, _SELU_SCALE * t, neg)


def _tc_messages(rb, sb, w, coef, W2p, b2, off, nblocks):
    BE = 4096
    ne = nblocks * BE
    grid = (nblocks,)

    def body(rb_ref, sb_ref, w_ref, coef_ref, w2_ref, b2_ref, out_ref):
        cf = coef_ref[...]
        c0 = cf[0, :16][:, None]
        dr = cf[1, :16][:, None]
        ds = cf[2, :16][:, None]
        vv = cf[3, :16][:, None]
        rbb = rb_ref[...][None, :]
        sbb = sb_ref[...][None, :]
        ww = w_ref[...][None, :]
        x_pre = c0 + dr * rbb + ds * sbb + vv * ww
        xT = _selu_lean(x_pre)
        y = jax.lax.dot_general(
            xT, w2_ref[...], (((0,), (0,)), ((), ())),
            preferred_element_type=jnp.float32,
        )
        out_ref[...] = _selu_lean(y + b2_ref[...][None, :])

    return pl.pallas_call(
        body,
        grid=grid,
        in_specs=[
            pl.BlockSpec((BE,), lambda i: (i + off,)),
            pl.BlockSpec((BE,), lambda i: (i + off,)),
            pl.BlockSpec((BE,), lambda i: (i + off,)),
            pl.BlockSpec((8, 128), lambda i: (0, 0)),
            pl.BlockSpec((16, FEAT), lambda i: (0, 0)),
            pl.BlockSpec((FEAT,), lambda i: (0,)),
        ],
        out_specs=pl.BlockSpec((BE, FEAT), lambda i: (i, 0)),
        out_shape=jax.ShapeDtypeStruct((ne, FEAT), jnp.float32),
        compiler_params=pltpu.CompilerParams(fuse_transposed_lhs_in_matmul=True),
    )(rb, sb, w, coef, W2p, b2)


def _sc_scatter(msg, recv3d, zeros, ne):
    ROWS = N_NODES // NS
    epw = ne // NW
    nchunk = epw // CHUNK

    NBUF = 5
    AHEAD = 3
    INFLIGHT = 2

    @functools.partial(
        pl.kernel,
        out_type=jax.ShapeDtypeStruct((NC * N_NODES, FEAT), jnp.float32),
        mesh=_sc_mesh(),
        scratch_types=[
            pltpu.VMEM((nchunk, CHUNK), jnp.int32),
            pltpu.VMEM((NBUF, CHUNK, FEAT), jnp.float32),
            pltpu.VMEM_SHARED((N_NODES, FEAT), jnp.float32),
            [pltpu.SemaphoreType.DMA] * NBUF,
            [pltpu.SemaphoreType.DMA] * NBUF,
            pltpu.SemaphoreType.DMA,
        ],
    )
    def k(msg_hbm, idx_hbm, zero_hbm, out_hbm, idx_v, msg_v,
          agg_sh, fsems, ssems, wsem):
        c = lax.axis_index("c")
        s = lax.axis_index("s")
        wid = s * NC + c
        base = wid * epw

        def fetch(j):
            return pltpu.async_copy(
                msg_hbm.at[pl.ds(base + j * CHUNK, CHUNK)],
                msg_v.at[j % NBUF], fsems[j % NBUF])

        def scat(j):
            return pltpu.async_copy(
                msg_v.at[j % NBUF], agg_sh.at[idx_v.at[j]],
                ssems[j % NBUF], add=True)

        fds = [None] * NBUF
        sds = [None] * NBUF
        for j in range(AHEAD):
            fds[j] = fetch(j)
        idx_cp = pltpu.async_copy(idx_hbm.at[wid], idx_v, wsem)

        pltpu.sync_copy(zero_hbm, msg_v.at[NBUF - 1])
        pltpu.sync_copy(msg_v.at[NBUF - 1], agg_sh.at[pl.ds(s * ROWS, CHUNK)])
        pltpu.sync_copy(msg_v.at[NBUF - 1],
                        agg_sh.at[pl.ds(s * ROWS + CHUNK, CHUNK)])
        idx_cp.wait()
        plsc.subcore_barrier()

        for j in range(nchunk):
            fds[j % NBUF].wait()
            sds[j % NBUF] = scat(j)
            if j >= INFLIGHT:
                sds[(j - INFLIGHT) % NBUF].wait()
            if j + AHEAD < nchunk:
                fds[(j + AHEAD) % NBUF] = fetch(j + AHEAD)
        for j in range(max(nchunk - INFLIGHT, 0), nchunk):
            sds[j % NBUF].wait()
        plsc.subcore_barrier()

        obase = c * N_NODES + s * ROWS
        pltpu.sync_copy(agg_sh.at[pl.ds(s * ROWS, CHUNK)], msg_v.at[0])
        w0 = pltpu.async_copy(msg_v.at[0], out_hbm.at[pl.ds(obase, CHUNK)],
                              fsems[0])
        pltpu.sync_copy(agg_sh.at[pl.ds(s * ROWS + CHUNK, CHUNK)], msg_v.at[1])
        pltpu.sync_copy(msg_v.at[1], out_hbm.at[pl.ds(obase + CHUNK, CHUNK)])
        w0.wait()

    return k(msg, recv3d, zeros)


def _tc_readout(p0, p1, W_a1, b_a1, W_a2, b_a2):
    BJ = 512
    NB = N_NODES // BJ
    grid = (2 * NB,)

    def body(p0_ref, p1_ref, w1_ref, b1_ref, w2_ref, b2_ref, out_ref,
             hs_ref, z_ref):
        i = pl.program_id(0)

        @pl.when(i == 0)
        def _():
            agg = (p0_ref[: N_NODES, :] + p0_ref[N_NODES:, :]
                   + p1_ref[: N_NODES, :] + p1_ref[N_NODES:, :])
            hs_ref[...] = jnp.sum(jnp.maximum(agg, 0.0), axis=1)
            out_ref[...] = jnp.zeros((1, 1), jnp.float32)

        @pl.when(i < NB)
        def _():
            y = jnp.dot(hs_ref[...][None, :], w1_ref[...],
                        preferred_element_type=jnp.float32)
            z_ref[i, :] = _selu_lean(y[0] + b1_ref[...])

        @pl.when(i >= NB)
        def _():
            y = jnp.zeros((1, BJ), jnp.float32)
            for kk in range(NB):
                y += jnp.dot(z_ref[kk, :][None, :],
                             w2_ref[pl.ds(kk * BJ, BJ), :],
                             preferred_element_type=jnp.float32)
            zz = _selu_lean(y[0] + b2_ref[...])
            out_ref[...] += jnp.sum(zz).reshape(1, 1)

    return pl.pallas_call(
        body,
        grid=grid,
        in_specs=[
            pl.BlockSpec((NC * N_NODES, FEAT), lambda i: (0, 0)),
            pl.BlockSpec((NC * N_NODES, FEAT), lambda i: (0, 0)),
            pl.BlockSpec((N_NODES, BJ), lambda i: (0, jnp.minimum(i, NB - 1))),
            pl.BlockSpec((BJ,), lambda i: (jnp.minimum(i, NB - 1),)),
            pl.BlockSpec((N_NODES, BJ), lambda i: (0, jnp.maximum(i - NB, 0))),
            pl.BlockSpec((BJ,), lambda i: (jnp.maximum(i - NB, 0),)),
        ],
        out_specs=pl.BlockSpec((1, 1), lambda i: (0, 0)),
        out_shape=jax.ShapeDtypeStruct((1, 1), jnp.float32),
        scratch_shapes=[
            pltpu.VMEM((N_NODES,), jnp.float32),
            pltpu.VMEM((NB, BJ), jnp.float32),
        ],
    )(p0, p1, W_a1, b_a1, W_a2, b_a2)


def kernel(h, senders, receivers, edge_weights, embed, W_m1, b_m1, W_m2, b_m2,
           Wq, bq, Wk, bk, W_a1, b_a1, W_a2, b_a2):
    A = W_m1[:5]
    B = W_m1[5:10]
    v = W_m1[10]
    e0 = embed[0]
    e1 = embed[1]
    c0 = e0 @ A + e0 @ B + b_m1
    dr = (e1 - e0) @ A
    ds = (e1 - e0) @ B

    coef = jnp.zeros((8, 128), jnp.float32)
    coef = coef.at[0, :11].set(c0)
    coef = coef.at[1, :11].set(dr)
    coef = coef.at[2, :11].set(ds)
    coef = coef.at[3, :11].set(v)
    W2p = jnp.zeros((16, FEAT), jnp.float32).at[:11, :].set(W_m2)

    hf = h.astype(jnp.float32)
    snd = senders.astype(jnp.int32)
    rcv = receivers.astype(jnp.int32)

    rb, sb = _sc_gather_bits(hf, snd, rcv)
    SPLITS = (28, 36)
    zeros = jnp.zeros((CHUNK, FEAT), jnp.float32)
    parts = []
    off = 0
    for nb in SPLITS:
        ne = nb * 4096
        msg_h = _tc_messages(rb, sb, edge_weights, coef, W2p, b_m2, off, nb)
        rcv_h = lax.dynamic_slice_in_dim(rcv, off * 4096, ne)
        parts.append(_sc_scatter(
            msg_h, rcv_h.reshape(NW, ne // NW // CHUNK, CHUNK), zeros, ne))
        off += nb
    res = _tc_readout(parts[0], parts[1], W_a1, b_a1, W_a2, b_a2)
    return res[0, 0]


# speedup vs baseline: 1.0581x; 1.0122x over previous
import functools

import jax
import jax.numpy as jnp
from jax import lax
from jax.experimental import pallas as pl
from jax.experimental.pallas import tpu as pltpu
from jax.experimental.pallas import tpu_sc as plsc

N_NODES = 4096
N_EDGES = 262144
FEAT = 128

NC = 2
NS = 16
NW = NC * NS
EPW = N_EDGES // NW
CHUNK = 128
NCHUNK = EPW // CHUNK

_SELU_SCALE = 1.0507009873554804934193349852946
_SELU_ALPHA = 1.6732632423543772848170429916717


def _selu(x):
    return _SELU_SCALE * jnp.where(x > 0, x, _SELU_ALPHA * (jnp.exp(x) - 1.0))


def _sc_mesh():
    return plsc.VectorSubcoreMesh(
        core_axis_name="c", subcore_axis_name="s", num_cores=NC, num_subcores=NS
    )


def _sc_gather_bits(hf, senders, receivers):
    @functools.partial(
        pl.kernel,
        out_type=(
            jax.ShapeDtypeStruct((N_EDGES,), jnp.float32),
            jax.ShapeDtypeStruct((N_EDGES,), jnp.float32),
        ),
        mesh=_sc_mesh(),
        compiler_params=pltpu.CompilerParams(needs_layout_passes=False),
        scratch_types=[
            pltpu.VMEM((N_NODES,), jnp.float32),
            pltpu.VMEM((EPW,), jnp.int32),
            pltpu.VMEM((EPW,), jnp.int32),
            pltpu.VMEM((EPW,), jnp.float32),
            pltpu.VMEM((EPW,), jnp.float32),
        ],
    )
    def k(hf_hbm, snd_hbm, rcv_hbm, rb_hbm, sb_hbm, hf_v, si_v, ri_v, rb_v, sb_v):
        wid = lax.axis_index("s") * NC + lax.axis_index("c")
        base = wid * EPW
        pltpu.sync_copy(hf_hbm, hf_v)
        pltpu.sync_copy(snd_hbm.at[pl.ds(base, EPW)], si_v)
        pltpu.sync_copy(rcv_hbm.at[pl.ds(base, EPW)], ri_v)

        def body(i, _):
            for u in range(8):
                off = i * 128 + u * 16
                ridx = ri_v[pl.ds(off, 16)]
                sidx = si_v[pl.ds(off, 16)]
                rb_v[pl.ds(off, 16)] = plsc.load_gather(hf_v, [ridx])
                sb_v[pl.ds(off, 16)] = plsc.load_gather(hf_v, [sidx])
            return 0

        lax.fori_loop(0, EPW // 128, body, 0)
        pltpu.sync_copy(rb_v, rb_hbm.at[pl.ds(base, EPW)])
        pltpu.sync_copy(sb_v, sb_hbm.at[pl.ds(base, EPW)])

    return k(hf, senders, receivers)


def _selu_lean(t):
    sa = _SELU_SCALE * _SELU_ALPHA
    neg = sa * jnp.exp(t) - sa
    return jnp.where(t > 0, _SELU_SCALE * t, neg)


def _tc_messages(rb, sb, w, coef, W2p, b2, off, nblocks):
    BE = 4096
    ne = nblocks * BE
    grid = (nblocks,)

    def body(rb_ref, sb_ref, w_ref, coef_ref, w2_ref, b2_ref, out_ref):
        cf = coef_ref[...]
        c0 = cf[0, :16][:, None]
        dr = cf[1, :16][:, None]
        ds = cf[2, :16][:, None]
        vv = cf[3, :16][:, None]
        rbb = rb_ref[...][None, :]
        sbb = sb_ref[...][None, :]
        ww = w_ref[...][None, :]
        x_pre = c0 + dr * rbb + ds * sbb + vv * ww
        xT = _selu_lean(x_pre)
        y = jax.lax.dot_general(
            xT, w2_ref[...], (((0,), (0,)), ((), ())),
            preferred_element_type=jnp.float32,
        )
        out_ref[...] = _selu_lean(y + b2_ref[...][None, :])

    return pl.pallas_call(
        body,
        grid=grid,
        in_specs=[
            pl.BlockSpec((BE,), lambda i: (i + off,)),
            pl.BlockSpec((BE,), lambda i: (i + off,)),
            pl.BlockSpec((BE,), lambda i: (i + off,)),
            pl.BlockSpec((8, 128), lambda i: (0, 0)),
            pl.BlockSpec((16, FEAT), lambda i: (0, 0)),
            pl.BlockSpec((FEAT,), lambda i: (0,)),
        ],
        out_specs=pl.BlockSpec((BE, FEAT), lambda i: (i, 0)),
        out_shape=jax.ShapeDtypeStruct((ne, FEAT), jnp.float32),
        compiler_params=pltpu.CompilerParams(fuse_transposed_lhs_in_matmul=True),
    )(rb, sb, w, coef, W2p, b2)


def _sc_scatter(msg, recv3d, zeros, ne):
    ROWS = N_NODES // NS
    epw = ne // NW
    nchunk = epw // CHUNK

    NBUF = 5
    AHEAD = 3
    INFLIGHT = 2

    @functools.partial(
        pl.kernel,
        out_type=jax.ShapeDtypeStruct((NC * N_NODES, FEAT), jnp.float32),
        mesh=_sc_mesh(),
        scratch_types=[
            pltpu.VMEM((nchunk, CHUNK), jnp.int32),
            pltpu.VMEM((NBUF, CHUNK, FEAT), jnp.float32),
            pltpu.VMEM_SHARED((N_NODES, FEAT), jnp.float32),
            [pltpu.SemaphoreType.DMA] * NBUF,
            [pltpu.SemaphoreType.DMA] * NBUF,
            pltpu.SemaphoreType.DMA,
        ],
    )
    def k(msg_hbm, idx_hbm, zero_hbm, out_hbm, idx_v, msg_v,
          agg_sh, fsems, ssems, wsem):
        c = lax.axis_index("c")
        s = lax.axis_index("s")
        wid = s * NC + c
        base = wid * epw

        def fetch(j):
            return pltpu.async_copy(
                msg_hbm.at[pl.ds(base + j * CHUNK, CHUNK)],
                msg_v.at[j % NBUF], fsems[j % NBUF])

        def scat(j):
            return pltpu.async_copy(
                msg_v.at[j % NBUF], agg_sh.at[idx_v.at[j]],
                ssems[j % NBUF], add=True)

        fds = [None] * NBUF
        sds = [None] * NBUF
        for j in range(AHEAD):
            fds[j] = fetch(j)
        idx_cp = pltpu.async_copy(idx_hbm.at[wid], idx_v, wsem)

        pltpu.sync_copy(zero_hbm, msg_v.at[NBUF - 1])
        pltpu.sync_copy(msg_v.at[NBUF - 1], agg_sh.at[pl.ds(s * ROWS, CHUNK)])
        pltpu.sync_copy(msg_v.at[NBUF - 1],
                        agg_sh.at[pl.ds(s * ROWS + CHUNK, CHUNK)])
        idx_cp.wait()
        plsc.subcore_barrier()

        for j in range(nchunk):
            fds[j % NBUF].wait()
            sds[j % NBUF] = scat(j)
            if j >= INFLIGHT:
                sds[(j - INFLIGHT) % NBUF].wait()
            if j + AHEAD < nchunk:
                fds[(j + AHEAD) % NBUF] = fetch(j + AHEAD)
        for j in range(max(nchunk - INFLIGHT, 0), nchunk):
            sds[j % NBUF].wait()
        plsc.subcore_barrier()

        obase = c * N_NODES + s * ROWS
        pltpu.sync_copy(agg_sh.at[pl.ds(s * ROWS, CHUNK)], msg_v.at[0])
        w0 = pltpu.async_copy(msg_v.at[0], out_hbm.at[pl.ds(obase, CHUNK)],
                              fsems[0])
        pltpu.sync_copy(agg_sh.at[pl.ds(s * ROWS + CHUNK, CHUNK)], msg_v.at[1])
        pltpu.sync_copy(msg_v.at[1], out_hbm.at[pl.ds(obase + CHUNK, CHUNK)])
        w0.wait()

    return k(msg, recv3d, zeros)


def _tc_readout(p0, p1, W_a1, b_a1, W_a2, b_a2):
    BJ = 512
    NB = N_NODES // BJ
    grid = (2 * NB,)

    def body(p0_ref, p1_ref, w1_ref, b1_ref, w2_ref, b2_ref, out_ref,
             hs_ref, z_ref):
        i = pl.program_id(0)

        @pl.when(i == 0)
        def _():
            agg = (p0_ref[: N_NODES, :] + p0_ref[N_NODES:, :]
                   + p1_ref[: N_NODES, :] + p1_ref[N_NODES:, :])
            hs_ref[...] = jnp.sum(jnp.maximum(agg, 0.0), axis=1)
            out_ref[...] = jnp.zeros((1, 1), jnp.float32)

        @pl.when(i < NB)
        def _():
            y = jnp.dot(hs_ref[...][None, :], w1_ref[...],
                        preferred_element_type=jnp.float32)
            z_ref[i, :] = _selu_lean(y[0] + b1_ref[...])

        @pl.when(i >= NB)
        def _():
            y = jnp.zeros((1, BJ), jnp.float32)
            for kk in range(NB):
                y += jnp.dot(z_ref[kk, :][None, :],
                             w2_ref[pl.ds(kk * BJ, BJ), :],
                             preferred_element_type=jnp.float32)
            zz = _selu_lean(y[0] + b2_ref[...])
            out_ref[...] += jnp.sum(zz).reshape(1, 1)

    return pl.pallas_call(
        body,
        grid=grid,
        in_specs=[
            pl.BlockSpec((NC * N_NODES, FEAT), lambda i: (0, 0)),
            pl.BlockSpec((NC * N_NODES, FEAT), lambda i: (0, 0)),
            pl.BlockSpec((N_NODES, BJ), lambda i: (0, jnp.minimum(i, NB - 1))),
            pl.BlockSpec((BJ,), lambda i: (jnp.minimum(i, NB - 1),)),
            pl.BlockSpec((N_NODES, BJ), lambda i: (0, jnp.maximum(i - NB, 0))),
            pl.BlockSpec((BJ,), lambda i: (jnp.maximum(i - NB, 0),)),
        ],
        out_specs=pl.BlockSpec((1, 1), lambda i: (0, 0)),
        out_shape=jax.ShapeDtypeStruct((1, 1), jnp.float32),
        scratch_shapes=[
            pltpu.VMEM((N_NODES,), jnp.float32),
            pltpu.VMEM((NB, BJ), jnp.float32),
        ],
    )(p0, p1, W_a1, b_a1, W_a2, b_a2)


def kernel(h, senders, receivers, edge_weights, embed, W_m1, b_m1, W_m2, b_m2,
           Wq, bq, Wk, bk, W_a1, b_a1, W_a2, b_a2):
    A = W_m1[:5]
    B = W_m1[5:10]
    v = W_m1[10]
    e0 = embed[0]
    e1 = embed[1]
    c0 = e0 @ A + e0 @ B + b_m1
    dr = (e1 - e0) @ A
    ds = (e1 - e0) @ B

    coef = jnp.zeros((8, 128), jnp.float32)
    coef = coef.at[0, :11].set(c0)
    coef = coef.at[1, :11].set(dr)
    coef = coef.at[2, :11].set(ds)
    coef = coef.at[3, :11].set(v)
    W2p = jnp.zeros((16, FEAT), jnp.float32).at[:11, :].set(W_m2)

    hf = h.astype(jnp.float32)
    snd = senders.astype(jnp.int32)
    rcv = receivers.astype(jnp.int32)

    rb, sb = _sc_gather_bits(hf, snd, rcv)
    SPLITS = (30, 34)
    zeros = jnp.zeros((CHUNK, FEAT), jnp.float32)
    parts = []
    off = 0
    for nb in SPLITS:
        ne = nb * 4096
        msg_h = _tc_messages(rb, sb, edge_weights, coef, W2p, b_m2, off, nb)
        rcv_h = lax.dynamic_slice_in_dim(rcv, off * 4096, ne)
        parts.append(_sc_scatter(
            msg_h, rcv_h.reshape(NW, ne // NW // CHUNK, CHUNK), zeros, ne))
        off += nb
    res = _tc_readout(parts[0], parts[1], W_a1, b_a1, W_a2, b_a2)
    return res[0, 0]
